# Initial kernel scaffold; baseline (speedup 1.0000x reference)
import jax
import jax.numpy as jnp
from jax.experimental import pallas as pl


def kernel(x, edge_index, Wl0, bl0, Wr0, Wl1, bl1, Wr1, Wp1, bp1, Wp2, bp2):
    raise NotImplementedError("write your pallas kernel here")


# speedup vs baseline: 5.3496x; 5.3496x over previous
import jax
import jax.numpy as jnp
from jax import lax
from jax.experimental import pallas as pl
from jax.experimental.pallas import tpu as pltpu
from jax.experimental.pallas import tpu_sc as plsc

NC = 2
NS = 16
NW = NC * NS


def _make_seg_sum(n_pad, e, f, with_cnt):
    edges_per_tile = e // NW
    assert edges_per_tile * NW == e
    chunk = 8
    for c in (128, 120, 112, 104, 96, 88, 80, 72, 64, 56, 48, 40, 32, 24, 16, 8):
        if edges_per_tile % c == 0:
            chunk = c
            break
    nchunks = edges_per_tile // chunk
    rows_main = n_pad // NS
    assert rows_main % 8 == 0 and rows_main % 128 == 0
    zrows = 128
    zreps = rows_main // zrows

    out_type = [jax.ShapeDtypeStruct((NC, n_pad, f), jnp.float32)]
    scratch = [
        pltpu.VMEM((chunk,), jnp.int32),
        pltpu.VMEM((chunk,), jnp.int32),
        pltpu.VMEM((chunk, f), jnp.float32),
        pltpu.VMEM((zrows, f), jnp.float32),
        pltpu.VMEM_SHARED((n_pad, f), jnp.float32),
        pltpu.SemaphoreType.DMA,
    ]
    if with_cnt:
        out_type.append(jax.ShapeDtypeStruct((NW, 1, n_pad), jnp.float32))
        scratch.append(pltpu.VMEM((1, n_pad), jnp.float32))

    mesh = plsc.VectorSubcoreMesh(core_axis_name="c", subcore_axis_name="s")

    def body(feat_hbm, src_hbm, dst_hbm, *refs):
        if with_cnt:
            acc_out, cnt_out, src_v, dst_v, rows_v, zbuf, acc_sh, sem, hist = refs
        else:
            acc_out, src_v, dst_v, rows_v, zbuf, acc_sh, sem = refs
        core = lax.axis_index("c")
        sub = lax.axis_index("s")
        wid = core * NS + sub

        def zinit(i, _):
            for j in range(f // 16):
                zbuf[i, pl.ds(j * 16, 16)] = jnp.zeros((16,), jnp.float32)
            return 0
        lax.fori_loop(0, zrows, zinit, 0)
        rbase = sub * rows_main
        for r in range(zreps):
            pltpu.sync_copy(zbuf, acc_sh.at[pl.ds(rbase + r * zrows, zrows)])
        if with_cnt:
            def zhist(i, _):
                hist[0, pl.ds(i * 16, 16)] = jnp.zeros((16,), jnp.float32)
                return 0
            lax.fori_loop(0, n_pad // 16, zhist, 0)
        plsc.subcore_barrier()

        ebase = wid * edges_per_tile
        zi16 = jnp.zeros((16,), jnp.int32)

        def echunk(c, _):
            base = ebase + c * chunk
            pltpu.sync_copy(src_hbm.at[pl.ds(base, chunk)], src_v)
            pltpu.sync_copy(dst_hbm.at[pl.ds(base, chunk)], dst_v)
            gat = pltpu.async_copy(feat_hbm.at[src_v], rows_v, sem)
            if with_cnt:
                for j in range(chunk // 16):
                    d16 = dst_v[pl.ds(j * 16, 16)]
                    cnts, last = plsc.scan_count(d16)
                    plsc.addupdate_scatter(
                        hist, [zi16, d16], cnts.astype(jnp.float32), mask=last)
            gat.wait()
            pltpu.sync_copy(rows_v, acc_sh.at[dst_v], add=True)
            return 0
        lax.fori_loop(0, nchunks, echunk, 0)
        plsc.subcore_barrier()

        pltpu.sync_copy(acc_sh.at[pl.ds(rbase, rows_main)],
                        acc_out.at[core, pl.ds(rbase, rows_main)])
        if with_cnt:
            pltpu.sync_copy(hist, cnt_out.at[wid])

    return pl.kernel(
        body, out_type=out_type, mesh=mesh, scratch_types=scratch,
        compiler_params=pltpu.CompilerParams(needs_layout_passes=False))


def _dot_t(a, w):
    return lax.dot_general(a, w, (((1,), (1,)), ((), ())),
                           preferred_element_type=jnp.float32,
                           precision=lax.Precision.HIGHEST)


def _mean(aggp_ref, cntp_ref):
    agg = aggp_ref[0] + aggp_ref[1]
    hists = cntp_ref[:, 0, :]
    cnt = lax.dot_general(hists, jnp.ones((NW, 1), jnp.float32),
                          (((0,), (0,)), ((), ())),
                          preferred_element_type=jnp.float32,
                          precision=lax.Precision.HIGHEST)
    return jnp.where(cnt > 0, agg / jnp.maximum(cnt, 1.0), 0.0)


def _layer0_body(aggp_ref, cntp_ref, x_ref, wl_ref, bl_ref, wr_ref, o_ref):
    h = (_dot_t(_mean(aggp_ref, cntp_ref), wl_ref[...]) + bl_ref[...]
         + _dot_t(x_ref[...], wr_ref[...]))
    o_ref[...] = jnp.maximum(h, 0.0)


def _final_body(aggp_ref, cntp_ref, h_ref, wl_ref, bl_ref, wr_ref,
                wp1_ref, bp1_ref, wp2_ref, bp2_ref, o_ref):
    h1 = (_dot_t(_mean(aggp_ref, cntp_ref), wl_ref[...]) + bl_ref[...]
          + _dot_t(h_ref[...], wr_ref[...]))
    h1 = jnp.maximum(h1, 0.0)
    z = _dot_t(h1, wp1_ref[...]) + bp1_ref[...]
    z = _dot_t(z, wp2_ref[...]) + bp2_ref[...]
    m = jnp.max(z, axis=1, keepdims=True)
    zs = z - m
    o_ref[...] = zs - jnp.log(jnp.sum(jnp.exp(zs), axis=1, keepdims=True))


def _tc_call(body, n_pad, f, o, blk, wpattern):
    grid = (n_pad // blk,)
    in_specs = [
        pl.BlockSpec((NC, blk, f), lambda i: (0, i, 0)),
        pl.BlockSpec((NW, 1, blk), lambda i: (0, 0, i)),
        pl.BlockSpec((blk, f), lambda i: (i, 0)),
    ]
    for k in wpattern:
        if k == "W":
            in_specs.append(pl.BlockSpec((f, f), lambda i: (0, 0)))
        else:
            in_specs.append(pl.BlockSpec((1, f), lambda i: (0, 0)))
    return pl.pallas_call(
        body,
        grid=grid,
        in_specs=in_specs,
        out_specs=pl.BlockSpec((blk, o), lambda i: (i, 0)),
        out_shape=jax.ShapeDtypeStruct((n_pad, o), jnp.float32),
        compiler_params=pltpu.CompilerParams(
            dimension_semantics=("parallel",)),
    )


@jax.jit
def kernel(x, edge_index, Wl0, bl0, Wr0, Wl1, bl1, Wr1, Wp1, bp1, Wp2, bp2):
    n, d = x.shape
    e = edge_index.shape[1]
    src = edge_index[0]
    dst = edge_index[1]
    blk = 1280
    n_pad = -(-n // blk) * blk
    x_pad = jnp.pad(x, ((0, n_pad - n), (0, 0)))

    aggp0, cntp = _make_seg_sum(n_pad, e, d, with_cnt=True)(x, src, dst)

    h0 = _tc_call(_layer0_body, n_pad, d, d, blk, "WbW")(
        aggp0, cntp, x_pad, Wl0, bl0.reshape(1, -1), Wr0)

    res1 = _make_seg_sum(n_pad, e, d, with_cnt=False)(h0, src, dst)
    aggp1 = res1[0] if isinstance(res1, (tuple, list)) else res1

    out = _tc_call(_final_body, n_pad, d, Wp2.shape[0], blk, "WbWWbWb")(
        aggp1, cntp, h0, Wl1, bl1.reshape(1, -1), Wr1,
        Wp1, bp1.reshape(1, -1), Wp2, bp2.reshape(1, -1))
    return out[:n]
